# baseline (speedup 1.0000x reference)
import jax
import jax.numpy as jnp
from jax import lax
from jax.experimental import pallas as pl
from jax.experimental.pallas import tpu as pltpu
from jax.experimental.pallas import tpu_sc as plsc

N_TOK = 16384
DIM = 4096
LANES = 16
NC = 2
NS = 16
NW = NC * NS

SC_ROWS = 4096
ROWS_PER_W = SC_ROWS // NW
CHUNK = 4
N_CHUNK = ROWS_PER_W // CHUNK
NBUF = 4

TC_BLOCK = 512


def _sc_body(x_hbm, a_hbm, o_hbm, a_v, bufs, sis, sos):
    wid = lax.axis_index("s") * NC + lax.axis_index("c")
    base = wid * ROWS_PER_W
    pltpu.sync_copy(a_hbm, a_v)

    def in_slice(c):
        return x_hbm.at[pl.ds(base + c * CHUNK, CHUNK)]

    def out_slice(c):
        return o_hbm.at[pl.ds(base + c * CHUNK, CHUNK)]

    for b in range(2):
        pltpu.async_copy(in_slice(b), bufs[b], sis[b])

    def quad_body(c4, _):
        for b in range(NBUF):
            c = c4 * NBUF + b
            buf, si, so = bufs[b], sis[b], sos[b]
            b2 = (b + 2) % NBUF

            @pl.when(c + 2 < N_CHUNK)
            def _():
                @pl.when(c - 2 >= 0)
                def _():
                    pltpu.make_async_copy(bufs[b2], out_slice(c - 2), sos[b2]).wait()
                pltpu.async_copy(in_slice(c + 2), bufs[b2], sis[b2])

            pltpu.make_async_copy(in_slice(c), buf, si).wait()

            @plsc.parallel_loop(0, DIM // LANES, unroll=8)
            def col_body(k):
                a_reg = a_v[pl.ds(k * LANES, LANES)]
                for r in range(CHUNK):
                    buf[r, pl.ds(k * LANES, LANES)] = (
                        buf[r, pl.ds(k * LANES, LANES)] * a_reg
                    )

            pltpu.async_copy(buf, out_slice(c), so)
        return 0

    lax.fori_loop(0, N_CHUNK // NBUF, quad_body, 0)

    for b in range(NBUF):
        c = N_CHUNK - NBUF + b
        pltpu.make_async_copy(bufs[b], out_slice(c), sos[b]).wait()


def _tc_body(a_ref, x_ref, o_ref):
    o_ref[...] = x_ref[...] * a_ref[...]


def kernel(x, a):
    mesh = plsc.VectorSubcoreMesh(core_axis_name="c", subcore_axis_name="s")
    sc_out = pl.kernel(
        _sc_body,
        out_type=jax.ShapeDtypeStruct((SC_ROWS, DIM), jnp.float32),
        mesh=mesh,
        scratch_types=[
            pltpu.VMEM((DIM,), jnp.float32),
            [pltpu.VMEM((CHUNK, DIM), jnp.float32) for _ in range(NBUF)],
            [pltpu.SemaphoreType.DMA for _ in range(NBUF)],
            [pltpu.SemaphoreType.DMA for _ in range(NBUF)],
        ],
    )(x, a)

    a2 = a.reshape(1, DIM)
    tc_full = pl.pallas_call(
        _tc_body,
        grid=((N_TOK - SC_ROWS) // TC_BLOCK,),
        in_specs=[
            pl.BlockSpec((1, DIM), lambda i: (0, 0)),
            pl.BlockSpec((TC_BLOCK, DIM), lambda i: (SC_ROWS // TC_BLOCK + i, 0)),
        ],
        out_specs=pl.BlockSpec((TC_BLOCK, DIM), lambda i: (SC_ROWS // TC_BLOCK + i, 0)),
        out_shape=jax.ShapeDtypeStruct((N_TOK, DIM), jnp.float32),
        compiler_params=pltpu.CompilerParams(
            dimension_semantics=("parallel",),
        ),
    )(a2, x)

    return lax.dynamic_update_slice(tc_full, sc_out, (0, 0))


# speedup vs baseline: 1.3660x; 1.3660x over previous
import jax
import jax.numpy as jnp
from jax import lax
from jax.experimental import pallas as pl
from jax.experimental.pallas import tpu as pltpu

N_TOK = 16384
DIM = 4096
R = 512
NCH = N_TOK // R
NBUF = 3


def _scale_body(a_ref, x_hbm, o_hbm, inb, outb, isem, osem):
    def in_copy(c, b):
        return pltpu.make_async_copy(
            x_hbm.at[pl.ds(c * R, R)], inb.at[b], isem.at[b])

    def out_copy(c, b):
        return pltpu.make_async_copy(
            outb.at[b], o_hbm.at[pl.ds(c * R, R)], osem.at[b])

    for b in range(NBUF):
        in_copy(b, b).start()

    def step(c, _):
        b = lax.rem(c, NBUF)
        in_copy(c, b).wait()

        @pl.when(c >= NBUF)
        def _():
            out_copy(c - NBUF, b).wait()

        outb[b] = inb[b] * a_ref[...]
        out_copy(c, b).start()

        @pl.when(c + NBUF < NCH)
        def _():
            in_copy(c + NBUF, b).start()

        return 0

    lax.fori_loop(0, NCH, step, 0)

    for c in range(NCH - NBUF, NCH):
        out_copy(c, c % NBUF).wait()


def kernel(x, a):
    a2 = a.reshape(1, DIM)
    return pl.pallas_call(
        _scale_body,
        in_specs=[
            pl.BlockSpec((1, DIM), lambda: (0, 0)),
            pl.BlockSpec(memory_space=pl.ANY),
        ],
        out_specs=pl.BlockSpec(memory_space=pl.ANY),
        out_shape=jax.ShapeDtypeStruct((N_TOK, DIM), jnp.float32),
        scratch_shapes=[
            pltpu.VMEM((NBUF, R, DIM), jnp.float32),
            pltpu.VMEM((NBUF, R, DIM), jnp.float32),
            pltpu.SemaphoreType.DMA((NBUF,)),
            pltpu.SemaphoreType.DMA((NBUF,)),
        ],
        compiler_params=pltpu.CompilerParams(
            vmem_limit_bytes=100 * 1024 * 1024,
        ),
    )(a2, x)


# speedup vs baseline: 1.3672x; 1.0009x over previous
import jax
import jax.numpy as jnp
from jax import lax
from jax.experimental import pallas as pl
from jax.experimental.pallas import tpu as pltpu

N_TOK = 16384
DIM = 4096
R = 1024
NCH = N_TOK // R
NBUF = 3


def _scale_body(a_ref, x_hbm, o_hbm, buf, isem, osem):
    def in_copy(c, b):
        return pltpu.make_async_copy(
            x_hbm.at[pl.ds(c * R, R)], buf.at[b], isem.at[b])

    def out_copy(c, b):
        return pltpu.make_async_copy(
            buf.at[b], o_hbm.at[pl.ds(c * R, R)], osem.at[b])

    for b in range(NBUF):
        in_copy(b, b).start()

    def step(c, _):
        b = lax.rem(c, NBUF)
        in_copy(c, b).wait()
        buf[b] = buf[b] * a_ref[...]
        out_copy(c, b).start()

        @pl.when(jnp.logical_and(c >= 1, c + 2 < NCH))
        def _():
            b2 = lax.rem(c + 2, NBUF)
            out_copy(c - 1, b2).wait()
            in_copy(c + 2, b2).start()

        return 0

    lax.fori_loop(0, NCH, step, 0)

    for c in range(NCH - NBUF, NCH):
        out_copy(c, c % NBUF).wait()


def kernel(x, a):
    a2 = a.reshape(1, DIM)
    return pl.pallas_call(
        _scale_body,
        in_specs=[
            pl.BlockSpec((1, DIM), lambda: (0, 0)),
            pl.BlockSpec(memory_space=pl.ANY),
        ],
        out_specs=pl.BlockSpec(memory_space=pl.ANY),
        out_shape=jax.ShapeDtypeStruct((N_TOK, DIM), jnp.float32),
        scratch_shapes=[
            pltpu.VMEM((NBUF, R, DIM), jnp.float32),
            pltpu.SemaphoreType.DMA((NBUF,)),
            pltpu.SemaphoreType.DMA((NBUF,)),
        ],
        compiler_params=pltpu.CompilerParams(
            vmem_limit_bytes=100 * 1024 * 1024,
        ),
    )(a2, x)


# speedup vs baseline: 1.3788x; 1.0084x over previous
import jax
import jax.numpy as jnp
from jax.experimental import pallas as pl
from jax.experimental.pallas import tpu as pltpu

N_TOK = 16384
DIM = 4096
BLOCK_ROWS = 1016


def _scale_body(a_ref, x_ref, o_ref):
    o_ref[...] = x_ref[...] * a_ref[...]


def kernel(x, a):
    a2 = a.reshape(1, DIM)
    grid = (pl.cdiv(N_TOK, BLOCK_ROWS),)
    return pl.pallas_call(
        _scale_body,
        grid=grid,
        in_specs=[
            pl.BlockSpec((1, DIM), lambda i: (0, 0)),
            pl.BlockSpec((BLOCK_ROWS, DIM), lambda i: (i, 0)),
        ],
        out_specs=pl.BlockSpec((BLOCK_ROWS, DIM), lambda i: (i, 0)),
        out_shape=jax.ShapeDtypeStruct((N_TOK, DIM), jnp.float32),
        compiler_params=pltpu.CompilerParams(
            dimension_semantics=("parallel",),
            vmem_limit_bytes=100 * 1024 * 1024,
        ),
    )(a2, x)
